# baseline (speedup 1.0000x reference)
import functools

import jax
import jax.numpy as jnp
import numpy as np
from jax import lax
from jax.experimental import pallas as pl
from jax.experimental.pallas import tpu as pltpu
from jax.experimental.pallas import tpu_sc as plsc

_DIM = 3
_N_LEVELS = 16
_N_FEATS = 2
_TABLE_SIZE = 2 ** 15
_BASE_RES = 16
_FINEST_RES = 512
_RANGE = 1.0

_P1 = 2654435761
_P2 = 805459861

_NC = 2
_NS = 16
_NW = _NC * _NS

_TP = 2048
_NTASK_PER_W = 16
_XRC = 2048
_LANES = 16
_UNROLL = 4
_TRB = 4096


def _resolutions_list():
    b = np.exp((np.log(_FINEST_RES) - np.log(_BASE_RES)) / (_N_LEVELS - 1))
    return [int(np.floor(_BASE_RES * (b ** l))) for l in range(_N_LEVELS)]


_RESS = _resolutions_list()


def _make_sc_kernel(npad):
    ntask = npad // _TP
    assert ntask == _NW * _NTASK_PER_W
    nvec = _TP // _LANES
    mesh = plsc.VectorSubcoreMesh(core_axis_name="c", subcore_axis_name="s")

    @functools.partial(
        pl.kernel,
        out_type=jax.ShapeDtypeStruct((npad * _N_LEVELS,), jnp.int32),
        mesh=mesh,
        compiler_params=pltpu.CompilerParams(needs_layout_passes=False),
        scratch_types=[
            pltpu.VMEM((_TABLE_SIZE,), jnp.int32),
            pltpu.VMEM((_TABLE_SIZE,), jnp.int32),
            pltpu.VMEM((_DIM, _TP), jnp.float32),
            pltpu.VMEM((_TP * _N_LEVELS,), jnp.int32),
            pltpu.SemaphoreType.DMA,
            pltpu.SemaphoreType.DMA,
        ],
    )
    def hashgrid_sc(x_hbm, tabs_hbm, out_hbm, tab_a, tab_b, x_v,
                    o_v, sem_a, sem_b):
        wid = lax.axis_index("s") * _NC + lax.axis_index("c")
        iota16 = lax.iota(jnp.int32, _LANES) * _N_LEVELS

        def task_body(tt, carry):
            t = wid * _NTASK_PER_W + tt
            _task_inner(t * _TP)
            return carry

        def _task_inner(tb):
            cps = {0: pltpu.async_copy(tabs_hbm.at[0], tab_a, sem_a)}

            pltpu.sync_copy(x_hbm.at[:, pl.ds(tb, _TP)], x_v)

            def xstep(i, c2):
                dst = i * _LANES
                for d in range(_DIM):
                    x_v[d, pl.ds(dst, _LANES)] = (
                        x_v[d, pl.ds(dst, _LANES)] + 1.0) * 0.5
                return c2
            lax.fori_loop(0, _TP // _LANES, xstep, 0)

            for l in range(_N_LEVELS):
                tab_v = tab_a if l % 2 == 0 else tab_b
                cps[l].wait()
                if l + 1 < _N_LEVELS:
                    nbuf = tab_b if l % 2 == 0 else tab_a
                    nsem = sem_b if l % 2 == 0 else sem_a
                    cps[l + 1] = pltpu.async_copy(
                        tabs_hbm.at[l + 1], nbuf, nsem)
                resf = float(_RESS[l])

                @plsc.parallel_loop(0, nvec, unroll=_UNROLL)
                def vstep(i, _resf=resf, _tab=tab_v, _l=l):
                    off = i * _LANES
                    x0 = x_v[0, pl.ds(off, _LANES)]
                    x1 = x_v[1, pl.ds(off, _LANES)]
                    x2 = x_v[2, pl.ds(off, _LANES)]
                    xs0 = x0 * _resf
                    xs1 = x1 * _resf
                    xs2 = x2 * _resf
                    xi0 = xs0.astype(jnp.int32)
                    xi1 = xs1.astype(jnp.int32)
                    xi2 = xs2.astype(jnp.int32)
                    xf0 = xs0 - xi0.astype(jnp.float32)
                    xf1 = xs1 - xi1.astype(jnp.float32)
                    xf2 = xs2 - xi2.astype(jnp.float32)
                    w0 = plsc.pack(xf0, xf0,
                                   format=plsc.PackFormat.INTERLEAVED)
                    w1 = plsc.pack(xf1, xf1,
                                   format=plsc.PackFormat.INTERLEAVED)
                    w2 = plsc.pack(xf2, xf2,
                                   format=plsc.PackFormat.INTERLEAVED)

                    c0a = xi0.astype(jnp.uint32)
                    c0b = c0a + jnp.uint32(1)
                    t1a = xi1.astype(jnp.uint32) * jnp.uint32(_P1)
                    t1b = t1a + jnp.uint32(_P1)
                    t2a = xi2.astype(jnp.uint32) * jnp.uint32(_P2)
                    t2b = t2a + jnp.uint32(_P2)

                    s = []
                    for t1 in (t1a, t1b):
                        for t2 in (t2a, t2b):
                            h12 = t1 ^ t2
                            ia = ((c0a ^ h12) & jnp.uint32(0x7FFF)
                                  ).astype(jnp.int32)
                            ib = ((c0b ^ h12) & jnp.uint32(0x7FFF)
                                  ).astype(jnp.int32)
                            fa = plsc.bitcast(
                                plsc.load_gather(_tab, [ia]), jnp.bfloat16)
                            fb = plsc.bitcast(
                                plsc.load_gather(_tab, [ib]), jnp.bfloat16)
                            s.append(fa + w0 * (fb - fa))
                    ua = s[0] + w1 * (s[2] - s[0])
                    ub = s[1] + w1 * (s[3] - s[1])
                    res = ua + w2 * (ub - ua)
                    resw = plsc.bitcast(res, jnp.int32)
                    plsc.store_scatter(
                        o_v, [iota16 + (off * _N_LEVELS + _l)], resw)

            pltpu.sync_copy(
                o_v, out_hbm.at[pl.ds(tb * _N_LEVELS, _TP * _N_LEVELS)])

        lax.fori_loop(0, _NTASK_PER_W, task_body, 0)

    return hashgrid_sc


def _convert_tc(y, npad):
    y128 = y.reshape(npad * _N_LEVELS // 128, 128)
    b = lax.bitcast_convert_type(y128, jnp.bfloat16)
    return b.astype(jnp.float32).reshape(npad, _N_LEVELS * _N_FEATS)


def kernel(x, tables):
    n = x.shape[0]
    block = _NW * _NTASK_PER_W * _TP
    npad = ((n + block - 1) // block) * block
    x_t = jnp.transpose(x)
    if npad != n:
        x_t = jnp.pad(x_t, ((0, 0), (0, npad - n)))
    y = _make_sc_kernel(npad)(x_t, _pack_tables(tables))
    return _convert_tc(y, npad)[:n]


def _pack_tables(tables):
    tb16 = tables.astype(jnp.bfloat16)
    return lax.bitcast_convert_type(tb16, jnp.int32)


# speedup vs baseline: 21.1721x; 21.1721x over previous
import functools

import jax
import jax.numpy as jnp
import numpy as np
from jax import lax
from jax.experimental import pallas as pl
from jax.experimental.pallas import tpu as pltpu
from jax.experimental.pallas import tpu_sc as plsc

_DIM = 3
_N_LEVELS = 16
_N_FEATS = 2
_TABLE_SIZE = 2 ** 15
_BASE_RES = 16
_FINEST_RES = 512
_RANGE = 1.0

_P1 = 2654435761
_P2 = 805459861

_NC = 2
_NS = 16
_NW = _NC * _NS

_TP = 2048
_NTASK_PER_W = 16
_XRC = 2048
_LANES = 16
_UNROLL = 4
_TRB = 2048


def _resolutions_list():
    b = np.exp((np.log(_FINEST_RES) - np.log(_BASE_RES)) / (_N_LEVELS - 1))
    return [int(np.floor(_BASE_RES * (b ** l))) for l in range(_N_LEVELS)]


_RESS = _resolutions_list()


def _make_sc_kernel(npad):
    ntask = npad // _TP
    assert ntask == _NW * _NTASK_PER_W
    nvec = _TP // _LANES
    mesh = plsc.VectorSubcoreMesh(core_axis_name="c", subcore_axis_name="s")

    @functools.partial(
        pl.kernel,
        out_type=jax.ShapeDtypeStruct((npad * _N_LEVELS,), jnp.int32),
        mesh=mesh,
        compiler_params=pltpu.CompilerParams(needs_layout_passes=False),
        scratch_types=[
            pltpu.VMEM((_TABLE_SIZE,), jnp.int32),
            pltpu.VMEM((_TABLE_SIZE,), jnp.int32),
            pltpu.VMEM((_DIM, _TP), jnp.float32),
            pltpu.VMEM((_TP * _N_LEVELS,), jnp.int32),
            pltpu.SemaphoreType.DMA,
            pltpu.SemaphoreType.DMA,
        ],
    )
    def hashgrid_sc(x_hbm, tabs_hbm, out_hbm, tab_a, tab_b, x_v,
                    o_v, sem_a, sem_b):
        wid = lax.axis_index("s") * _NC + lax.axis_index("c")
        iota16 = lax.iota(jnp.int32, _LANES) * _N_LEVELS

        def task_body(tt, carry):
            t = wid * _NTASK_PER_W + tt
            _task_inner(t * _TP)
            return carry

        def _task_inner(tb):
            cps = {0: pltpu.async_copy(tabs_hbm.at[0], tab_a, sem_a)}

            pltpu.sync_copy(x_hbm.at[:, pl.ds(tb, _TP)], x_v)

            def xstep(i, c2):
                dst = i * _LANES
                for d in range(_DIM):
                    x_v[d, pl.ds(dst, _LANES)] = (
                        x_v[d, pl.ds(dst, _LANES)] + 1.0) * 0.5
                return c2
            lax.fori_loop(0, _TP // _LANES, xstep, 0)

            for l in range(_N_LEVELS):
                tab_v = tab_a if l % 2 == 0 else tab_b
                cps[l].wait()
                if l + 1 < _N_LEVELS:
                    nbuf = tab_b if l % 2 == 0 else tab_a
                    nsem = sem_b if l % 2 == 0 else sem_a
                    cps[l + 1] = pltpu.async_copy(
                        tabs_hbm.at[l + 1], nbuf, nsem)
                resf = float(_RESS[l])

                @plsc.parallel_loop(0, nvec, unroll=_UNROLL)
                def vstep(i, _resf=resf, _tab=tab_v, _l=l):
                    off = i * _LANES
                    x0 = x_v[0, pl.ds(off, _LANES)]
                    x1 = x_v[1, pl.ds(off, _LANES)]
                    x2 = x_v[2, pl.ds(off, _LANES)]
                    xs0 = x0 * _resf
                    xs1 = x1 * _resf
                    xs2 = x2 * _resf
                    xi0 = xs0.astype(jnp.int32)
                    xi1 = xs1.astype(jnp.int32)
                    xi2 = xs2.astype(jnp.int32)
                    xf0 = xs0 - xi0.astype(jnp.float32)
                    xf1 = xs1 - xi1.astype(jnp.float32)
                    xf2 = xs2 - xi2.astype(jnp.float32)
                    w0 = plsc.pack(xf0, xf0,
                                   format=plsc.PackFormat.INTERLEAVED)
                    w1 = plsc.pack(xf1, xf1,
                                   format=plsc.PackFormat.INTERLEAVED)
                    w2 = plsc.pack(xf2, xf2,
                                   format=plsc.PackFormat.INTERLEAVED)

                    c0a = xi0.astype(jnp.uint32)
                    c0b = c0a + jnp.uint32(1)
                    t1a = xi1.astype(jnp.uint32) * jnp.uint32(_P1)
                    t1b = t1a + jnp.uint32(_P1)
                    t2a = xi2.astype(jnp.uint32) * jnp.uint32(_P2)
                    t2b = t2a + jnp.uint32(_P2)

                    s = []
                    for t1 in (t1a, t1b):
                        for t2 in (t2a, t2b):
                            h12 = t1 ^ t2
                            ia = ((c0a ^ h12) & jnp.uint32(0x7FFF)
                                  ).astype(jnp.int32)
                            ib = ((c0b ^ h12) & jnp.uint32(0x7FFF)
                                  ).astype(jnp.int32)
                            fa = plsc.bitcast(
                                plsc.load_gather(_tab, [ia]), jnp.bfloat16)
                            fb = plsc.bitcast(
                                plsc.load_gather(_tab, [ib]), jnp.bfloat16)
                            s.append(fa + w0 * (fb - fa))
                    ua = s[0] + w1 * (s[2] - s[0])
                    ub = s[1] + w1 * (s[3] - s[1])
                    res = ua + w2 * (ub - ua)
                    resw = plsc.bitcast(res, jnp.int32)
                    plsc.store_scatter(
                        o_v, [iota16 + (off * _N_LEVELS + _l)], resw)

            pltpu.sync_copy(
                o_v, out_hbm.at[pl.ds(tb * _N_LEVELS, _TP * _N_LEVELS)])

        lax.fori_loop(0, _NTASK_PER_W, task_body, 0)

    return hashgrid_sc


def _tr_body(i_ref, o_ref):
    o_ref[...] = jnp.transpose(i_ref[...], (1, 0))


def _transpose_tc(y, npad):
    return pl.pallas_call(
        _tr_body,
        grid=(npad // _TRB,),
        in_specs=[pl.BlockSpec((_N_LEVELS * _N_FEATS, _TRB),
                               lambda i: (0, i))],
        out_specs=pl.BlockSpec((_TRB, _N_LEVELS * _N_FEATS),
                               lambda i: (i, 0)),
        out_shape=jax.ShapeDtypeStruct((npad, _N_LEVELS * _N_FEATS),
                                       jnp.float32),
    )(y)


def kernel(x, tables):
    n = x.shape[0]
    block = _NW * _NTASK_PER_W * _TP
    npad = ((n + block - 1) // block) * block
    x_t = jnp.transpose(x)
    if npad != n:
        x_t = jnp.pad(x_t, ((0, 0), (0, npad - n)))
    y = _make_sc_kernel(npad)(x_t, _pack_tables(tables))
    yb = lax.bitcast_convert_type(y.reshape(npad, _N_LEVELS),
                                  jnp.bfloat16)
    return yb[:n].reshape(n, _N_LEVELS * _N_FEATS).astype(jnp.float32)


def _pack_tables(tables):
    tb16 = tables.astype(jnp.bfloat16)
    return lax.bitcast_convert_type(tb16, jnp.int32)
